# Initial kernel scaffold; baseline (speedup 1.0000x reference)
import jax
import jax.numpy as jnp
from jax.experimental import pallas as pl


def kernel(A, X, W1, W2):
    raise NotImplementedError("write your pallas kernel here")


# speedup vs baseline: 4.1145x; 4.1145x over previous
import functools

import jax
import jax.numpy as jnp
from jax import lax
from jax.experimental import pallas as pl
from jax.experimental.pallas import tpu as pltpu
from jax.experimental.pallas import tpu_sc as plsc

NC = 2
NS = 16
NW = NC * NS
CH = 128


def _sc_scatter_call(d, nchunk, n_pad):
    rpz = n_pad // NS
    zrep = rpz // CH

    mesh = plsc.VectorSubcoreMesh(
        core_axis_name="c", subcore_axis_name="s", num_cores=NC,
        num_subcores=NS)

    @functools.partial(
        pl.kernel,
        mesh=mesh,
        out_type=jax.ShapeDtypeStruct((NC, n_pad, d), jnp.float32),
        scratch_types=[
            pltpu.VMEM((nchunk, CH), jnp.int32),
            pltpu.VMEM((nchunk, CH), jnp.int32),
            pltpu.VMEM((CH, d), jnp.float32),
            pltpu.VMEM_SHARED((n_pad, d), jnp.float32),
            pltpu.SemaphoreType.DMA,
        ],
    )
    def scatter_kernel(sup_hbm, src_hbm, dst_hbm, out_hbm,
                       src_v, dst_v, rows_v, acc_sh, sem):
        c = lax.axis_index("c")
        s = lax.axis_index("s")
        wid = s * NC + c

        zero16 = jnp.zeros((16,), jnp.float32)

        def zrow(i, carry):
            for j in range(d // 16):
                rows_v[i, pl.ds(j * 16, 16)] = zero16
            return carry

        lax.fori_loop(0, CH, zrow, 0)
        for k in range(zrep):
            pltpu.sync_copy(rows_v, acc_sh.at[pl.ds(s * rpz + k * CH, CH)])
        plsc.subcore_barrier()

        pltpu.sync_copy(src_hbm.at[wid], src_v)
        pltpu.sync_copy(dst_hbm.at[wid], dst_v)

        def step(j, carry):
            pltpu.async_copy(sup_hbm.at[src_v.at[j]], rows_v, sem).wait()
            pltpu.sync_copy(rows_v, acc_sh.at[dst_v.at[j]], add=True)
            return carry

        lax.fori_loop(0, nchunk, step, 0)
        plsc.subcore_barrier()

        pltpu.sync_copy(acc_sh.at[pl.ds(s * rpz, rpz)],
                        out_hbm.at[c, pl.ds(s * rpz, rpz)])

    return scatter_kernel


def _matmul_call(x, w, rows_blk):
    n, d = x.shape

    def body(x_ref, w_ref, o_ref):
        o_ref[...] = jnp.dot(x_ref[...], w_ref[...],
                             preferred_element_type=jnp.float32)

    return pl.pallas_call(
        body,
        grid=(n // rows_blk,),
        in_specs=[
            pl.BlockSpec((rows_blk, d), lambda i: (i, 0)),
            pl.BlockSpec((d, d), lambda i: (0, 0)),
        ],
        out_specs=pl.BlockSpec((rows_blk, d), lambda i: (i, 0)),
        out_shape=jax.ShapeDtypeStruct((n, d), jnp.float32),
    )(x, w)


def _combine_matmul_call(p, w, rows_blk):
    _, n, d = p.shape

    def body(p_ref, w_ref, o_ref):
        h = jnp.maximum(p_ref[0] + p_ref[1], 0.0)
        o_ref[...] = jnp.dot(h, w_ref[...], preferred_element_type=jnp.float32)

    return pl.pallas_call(
        body,
        grid=(n // rows_blk,),
        in_specs=[
            pl.BlockSpec((NC, rows_blk, d), lambda i: (0, i, 0)),
            pl.BlockSpec((d, d), lambda i: (0, 0)),
        ],
        out_specs=pl.BlockSpec((rows_blk, d), lambda i: (i, 0)),
        out_shape=jax.ShapeDtypeStruct((n, d), jnp.float32),
    )(p, w)


def _combine_relu_call(p, rows_blk):
    _, n, d = p.shape

    def body(p_ref, o_ref):
        o_ref[...] = jnp.maximum(p_ref[0] + p_ref[1], 0.0)

    return pl.pallas_call(
        body,
        grid=(n // rows_blk,),
        in_specs=[pl.BlockSpec((NC, rows_blk, d), lambda i: (0, i, 0))],
        out_specs=pl.BlockSpec((rows_blk, d), lambda i: (i, 0)),
        out_shape=jax.ShapeDtypeStruct((n, d), jnp.float32),
    )(p)


def kernel(A, X, W1, W2):
    x = X[0]
    n, d = x.shape
    e = A.shape[1]

    epw = -(-e // (NW * CH)) * CH
    nchunk = epw // CH
    e_pad = NW * epw
    n_pad = -(-(n + 1) // (NS * CH)) * NS * CH

    src = jnp.concatenate(
        [A[0], jnp.zeros((e_pad - e,), jnp.int32)]).reshape(NW, nchunk, CH)
    dst = jnp.concatenate(
        [A[1], jnp.full((e_pad - e,), n, jnp.int32)]).reshape(NW, nchunk, CH)

    scatter = _sc_scatter_call(d, nchunk, n_pad)

    sup1 = _matmul_call(x, W1, 1000)
    p1 = scatter(sup1, src, dst)
    sup2 = _combine_matmul_call(p1, W2, 1024)
    p2 = scatter(sup2, src, dst)
    out = _combine_relu_call(p2, 1024)
    return out[None, :n, :]
